# baseline (speedup 1.0000x reference)
import functools

import jax
import jax.numpy as jnp
from jax import lax
from jax.experimental import pallas as pl
from jax.experimental.pallas import tpu as pltpu
from jax.experimental.pallas import tpu_sc as plsc

_NC = 2
_NS = 16
_NW = _NC * _NS

_CH = 128


@functools.partial(jax.jit, static_argnames=("bpw", "nch", "d"))
def _sc_gather(idx, table, *, bpw, nch, d):
    B = _NW * bpw
    mesh = plsc.VectorSubcoreMesh(core_axis_name="c", subcore_axis_name="s")

    @functools.partial(
        pl.kernel,
        mesh=mesh,
        compiler_params=pltpu.CompilerParams(use_tc_tiling_on_sc=False),
        out_type=jax.ShapeDtypeStruct((B, d), jnp.float32),
        scratch_types=[
            pltpu.VMEM((nch, _CH), jnp.int32),
            pltpu.VMEM((bpw, d), jnp.float32),
            pltpu.SemaphoreType.DMA,
        ],
    )
    def gather_kernel(idx_hbm, table_hbm, out_hbm, idx_v, rows_v, sem):
        wid = lax.axis_index("s") * _NC + lax.axis_index("c")
        base = wid * bpw
        pltpu.sync_copy(idx_hbm.at[wid], idx_v)
        copies = [
            pltpu.async_copy(
                table_hbm.at[idx_v.at[j]],
                rows_v.at[pl.ds(j * _CH, _CH)],
                sem,
            )
            for j in range(nch)
        ]
        for c in copies:
            c.wait()
        pltpu.sync_copy(rows_v, out_hbm.at[pl.ds(base, bpw)])

    return gather_kernel(idx, table)


def _relu_linear_body(x_ref, wt_ref, b_ref, o_ref):
    x = jnp.maximum(x_ref[...], 0.0)
    o_ref[...] = (
        jnp.dot(x, wt_ref[...], preferred_element_type=jnp.float32) + b_ref[...]
    )


def kernel(user_idx, embedding_table, linear_w, linear_b):
    B = user_idx.shape[0]
    V, D = embedding_table.shape
    F = linear_w.shape[0]

    bpw = B // _NW
    nch = bpw // _CH
    idx = user_idx.reshape(-1).astype(jnp.int32).reshape(_NW, nch, _CH)

    gathered = _sc_gather(idx, embedding_table, bpw=bpw, nch=nch, d=D)

    BM = 1024
    out2d = pl.pallas_call(
        _relu_linear_body,
        grid=(B // BM,),
        in_specs=[
            pl.BlockSpec((BM, D), lambda i: (i, 0)),
            pl.BlockSpec((D, F), lambda i: (0, 0)),
            pl.BlockSpec((1, F), lambda i: (0, 0)),
        ],
        out_specs=pl.BlockSpec((BM, F), lambda i: (i, 0)),
        out_shape=jax.ShapeDtypeStruct((B, F), jnp.float32),
    )(gathered, linear_w.T, linear_b.reshape(1, F))

    return out2d.reshape(B, 1, F)


# speedup vs baseline: 1.6863x; 1.6863x over previous
import functools

import jax
import jax.numpy as jnp
from jax import lax
from jax.experimental import pallas as pl
from jax.experimental.pallas import tpu as pltpu
from jax.experimental.pallas import tpu_sc as plsc

_NC = 2
_NS = 16
_NW = _NC * _NS

_CH = 128


@functools.partial(jax.jit, static_argnames=("bpw", "d"))
def _sc_gather(idx, table, *, bpw, d):
    B = _NW * bpw
    mesh = plsc.VectorSubcoreMesh(core_axis_name="c", subcore_axis_name="s")

    @functools.partial(
        pl.kernel,
        mesh=mesh,
        out_type=jax.ShapeDtypeStruct((B, d), jnp.float32),
        scratch_types=[
            pltpu.VMEM((bpw,), jnp.int32),
            pltpu.VMEM((bpw, d), jnp.float32),
            pltpu.SemaphoreType.DMA,
        ],
    )
    def gather_kernel(idx_hbm, table_hbm, out_hbm, idx_s, rows_v, sem):
        wid = lax.axis_index("s") * _NC + lax.axis_index("c")
        base = wid * bpw
        pltpu.sync_copy(idx_hbm.at[wid], idx_s)

        def issue(g, carry):
            iv = idx_s[pl.ds(g * 16, 16)]
            for k in range(16):
                pltpu.async_copy(
                    table_hbm.at[pl.ds(iv[k], 1)],
                    rows_v.at[pl.ds(g * 16 + k, 1)],
                    sem,
                )
            return carry

        lax.fori_loop(0, bpw // 16, issue, 0)
        pltpu.make_async_copy(out_hbm.at[pl.ds(base, bpw)], rows_v, sem).wait()
        pltpu.sync_copy(rows_v, out_hbm.at[pl.ds(base, bpw)])

    return gather_kernel(idx, table)


def _relu_linear_body(x_ref, wt_ref, b_ref, o_ref):
    x = jnp.maximum(x_ref[...], 0.0)
    o_ref[...] = (
        jnp.dot(x, wt_ref[...], preferred_element_type=jnp.float32) + b_ref[...]
    )


def kernel(user_idx, embedding_table, linear_w, linear_b):
    B = user_idx.shape[0]
    V, D = embedding_table.shape
    F = linear_w.shape[0]

    bpw = B // _NW
    idx = user_idx.reshape(-1).astype(jnp.int32).reshape(_NW, bpw)

    gathered = _sc_gather(idx, embedding_table, bpw=bpw, d=D)

    BM = 1024
    out2d = pl.pallas_call(
        _relu_linear_body,
        grid=(B // BM,),
        in_specs=[
            pl.BlockSpec((BM, D), lambda i: (i, 0)),
            pl.BlockSpec((D, F), lambda i: (0, 0)),
            pl.BlockSpec((1, F), lambda i: (0, 0)),
        ],
        out_specs=pl.BlockSpec((BM, F), lambda i: (i, 0)),
        out_shape=jax.ShapeDtypeStruct((B, F), jnp.float32),
    )(gathered, linear_w.T, linear_b.reshape(1, F))

    return out2d.reshape(B, 1, F)


# speedup vs baseline: 1.6871x; 1.0005x over previous
import functools

import jax
import jax.numpy as jnp
from jax import lax
from jax.experimental import pallas as pl
from jax.experimental.pallas import tpu as pltpu
from jax.experimental.pallas import tpu_sc as plsc

_NC = 2
_NS = 16
_NW = _NC * _NS

_CH = 128


@functools.partial(jax.jit, static_argnames=("bpw", "d"))
def _sc_gather(idx, table, *, bpw, d):
    B = _NW * bpw
    mesh = plsc.VectorSubcoreMesh(core_axis_name="c", subcore_axis_name="s")

    @functools.partial(
        pl.kernel,
        mesh=mesh,
        compiler_params=pltpu.CompilerParams(use_tc_tiling_on_sc=True),
        out_type=jax.ShapeDtypeStruct((B, d), jnp.float32),
        scratch_types=[
            pltpu.VMEM((bpw,), jnp.int32),
            pltpu.VMEM((bpw, d), jnp.float32),
            pltpu.SemaphoreType.DMA,
        ],
    )
    def gather_kernel(idx_hbm, table_hbm, out_hbm, idx_s, rows_v, sem):
        wid = lax.axis_index("s") * _NC + lax.axis_index("c")
        base = wid * bpw
        pltpu.sync_copy(idx_hbm.at[wid], idx_s)

        def issue(g, carry):
            iv = idx_s[pl.ds(g * 16, 16)]
            for k in range(16):
                pltpu.async_copy(
                    table_hbm.at[pl.ds(iv[k], 1)],
                    rows_v.at[pl.ds(g * 16 + k, 1)],
                    sem,
                )
            return carry

        lax.fori_loop(0, bpw // 16, issue, 0)
        pltpu.make_async_copy(out_hbm.at[pl.ds(base, bpw)], rows_v, sem).wait()
        pltpu.sync_copy(rows_v, out_hbm.at[pl.ds(base, bpw)])

    return gather_kernel(idx, table)


def _relu_linear_body(x_ref, wt_ref, b_ref, o_ref):
    x = jnp.maximum(x_ref[...], 0.0)
    o_ref[...] = (
        jnp.dot(x, wt_ref[...], preferred_element_type=jnp.float32) + b_ref[...]
    )


def kernel(user_idx, embedding_table, linear_w, linear_b):
    B = user_idx.shape[0]
    V, D = embedding_table.shape
    F = linear_w.shape[0]

    bpw = B // _NW
    idx = user_idx.reshape(-1).astype(jnp.int32).reshape(_NW, bpw)

    gathered = _sc_gather(idx, embedding_table, bpw=bpw, d=D)

    BM = 1024
    out2d = pl.pallas_call(
        _relu_linear_body,
        grid=(B // BM,),
        in_specs=[
            pl.BlockSpec((BM, D), lambda i: (i, 0)),
            pl.BlockSpec((D, F), lambda i: (0, 0)),
            pl.BlockSpec((1, F), lambda i: (0, 0)),
        ],
        out_specs=pl.BlockSpec((BM, F), lambda i: (i, 0)),
        out_shape=jax.ShapeDtypeStruct((B, F), jnp.float32),
    )(gathered, linear_w.T, linear_b.reshape(1, F))

    return out2d.reshape(B, 1, F)
